# baseline (speedup 1.0000x reference)
import functools

import jax
import jax.numpy as jnp
import numpy as np
from jax import lax
from jax.experimental import pallas as pl
from jax.experimental.pallas import tpu as pltpu
from jax.experimental.pallas import tpu_sc as plsc

_EMB_LIST = [100, 11, 11, 11, 9, 4, 9, 5, 4, 8]
_TOTAL = 172
_NFEAT = 10
_OFFSETS = [int(x) for x in np.concatenate([[0], np.cumsum(_EMB_LIST)[:-1]])]

_N = 100000
_CHUNK = 160
_NCHUNKS = _N // _CHUNK
_NGROUPS = _CHUNK // 16
_NW = 16


def _sc_body(atom_hbm, out_hbm, atom_v, out_v):
    wid = lax.axis_index("s")
    lanes = lax.broadcasted_iota(jnp.int32, (16,), 0)
    ones = jnp.full((16,), 1.0, dtype=jnp.float32)
    zeros = jnp.zeros((16,), dtype=jnp.float32)

    def zero_body(k, c):
        flat = k * 16 + lanes
        plsc.store_scatter(out_v, [flat // _TOTAL, flat % _TOTAL], zeros)
        return c

    lax.fori_loop(0, _CHUNK * _TOTAL // 16, zero_body, 0)

    def scatter_chunk(value):
        def group_body(g, c):
            rows = g * 16 + lanes
            for i in range(_NFEAT):
                vals = plsc.load_gather(
                    atom_v, [rows, jnp.full((16,), i, dtype=jnp.int32)]
                )
                mask = (vals >= 0) & (vals < _EMB_LIST[i])
                plsc.store_scatter(
                    out_v, [rows, vals + _OFFSETS[i]], value, mask=mask
                )
            return c

        lax.fori_loop(0, _NGROUPS, group_body, 0)

    n_mine = (_NCHUNKS - 1 - wid) // _NW + 1

    def chunk_body(j, carry):
        r0 = (wid + j * _NW) * _CHUNK
        pltpu.sync_copy(atom_hbm.at[pl.ds(r0, _CHUNK), :], atom_v)
        scatter_chunk(ones)
        pltpu.sync_copy(out_v, out_hbm.at[pl.ds(r0, _CHUNK), :])
        scatter_chunk(zeros)
        return carry

    lax.fori_loop(0, n_mine, chunk_body, 0)


@jax.jit
def kernel(atom):
    mesh = plsc.VectorSubcoreMesh(
        core_axis_name="c", subcore_axis_name="s", num_cores=1
    )
    run = pl.kernel(
        _sc_body,
        out_type=jax.ShapeDtypeStruct((_N, _TOTAL), jnp.float32),
        mesh=mesh,
        scratch_types=[
            pltpu.VMEM((_CHUNK, _NFEAT), jnp.int32),
            pltpu.VMEM((_CHUNK, _TOTAL), jnp.float32),
        ],
        compiler_params=pltpu.CompilerParams(needs_layout_passes=False),
    )
    return run(atom.astype(jnp.int32))


# speedup vs baseline: 1.9799x; 1.9799x over previous
import functools

import jax
import jax.numpy as jnp
import numpy as np
from jax import lax
from jax.experimental import pallas as pl
from jax.experimental.pallas import tpu as pltpu
from jax.experimental.pallas import tpu_sc as plsc

_EMB_LIST = [100, 11, 11, 11, 9, 4, 9, 5, 4, 8]
_TOTAL = 172
_NFEAT = 10
_OFFSETS = [int(x) for x in np.concatenate([[0], np.cumsum(_EMB_LIST)[:-1]])]

_N = 100000
_CHUNK = 160
_NCHUNKS = _N // _CHUNK
_NGROUPS = _CHUNK // 16
_TAIL = _CHUNK - _NGROUPS * 16
_NSPANS = _TOTAL // 16
_TAILSPAN = _TOTAL - _NSPANS * 16
_NW = 32


def _sc_body(
    atom_hbm,
    out_hbm,
    atom_v0,
    atom_v1,
    out_v0,
    out_v1,
    sem_i0,
    sem_i1,
    sem_o0,
    sem_o1,
):
    wid = lax.axis_index("s") * 2 + lax.axis_index("c")
    lanes = lax.broadcasted_iota(jnp.int32, (16,), 0)
    ones = jnp.full((16,), 1.0, dtype=jnp.float32)
    zeros = jnp.zeros((16,), dtype=jnp.float32)
    tail_mask = lanes < _TAILSPAN

    atom_bufs = (atom_v0, atom_v1)
    out_bufs = (out_v0, out_v1)
    in_sems = (sem_i0, sem_i1)
    out_sems = (sem_o0, sem_o1)

    n_mine = (_NCHUNKS - 1 - wid) // _NW + 1

    def row0(j):
        return (wid + j * _NW) * _CHUNK

    def zero_buf(out_v):
        def zrow(r, c):
            for k in range(_NSPANS):
                out_v[r, pl.ds(k * 16, 16)] = zeros
            plsc.store_scatter(
                out_v,
                [jnp.full((16,), r, dtype=jnp.int32), _NSPANS * 16 + lanes],
                zeros,
                mask=tail_mask,
            )
            return c

        lax.fori_loop(0, _CHUNK, zrow, 0)

    def scatter_buf(atom_v, out_v):
        def group_body(g, c):
            rows = g * 16 + lanes
            gmask = rows < _CHUNK
            for i in range(_NFEAT):
                vals = plsc.load_gather(
                    atom_v,
                    [rows, jnp.full((16,), i, dtype=jnp.int32)],
                    mask=gmask,
                )
                mask = gmask & (vals >= 0) & (vals < _EMB_LIST[i])
                plsc.store_scatter(
                    out_v, [rows, vals + _OFFSETS[i]], ones, mask=mask
                )
            return c

        lax.fori_loop(0, _NGROUPS + (1 if _TAIL else 0), group_body, 0)

    @pl.when(n_mine > 0)
    def _():
        pltpu.async_copy(
            atom_hbm.at[pl.ds(row0(0), _CHUNK), :], atom_bufs[0], in_sems[0]
        )

    npairs = (n_mine + 1) // 2

    def pair_body(j2, carry):
        for b in range(2):
            j = j2 * 2 + b

            @pl.when(j < n_mine)
            def _():
                r0 = row0(j)
                @pl.when(j >= 2)
                def _():
                    pltpu.make_async_copy(
                        out_bufs[b],
                        out_hbm.at[pl.ds(r0, _CHUNK), :],
                        out_sems[b],
                    ).wait()

                zero_buf(out_bufs[b])
                pltpu.make_async_copy(
                    atom_hbm.at[pl.ds(r0, _CHUNK), :],
                    atom_bufs[b],
                    in_sems[b],
                ).wait()
                scatter_buf(atom_bufs[b], out_bufs[b])
                pltpu.async_copy(
                    out_bufs[b],
                    out_hbm.at[pl.ds(r0, _CHUNK), :],
                    out_sems[b],
                )

                @pl.when(j + 1 < n_mine)
                def _():
                    pltpu.async_copy(
                        atom_hbm.at[pl.ds(row0(j + 1), _CHUNK), :],
                        atom_bufs[1 - b],
                        in_sems[1 - b],
                    )

        return carry

    lax.fori_loop(0, npairs, pair_body, 0)

    for b in range(2):

        @pl.when(n_mine > b)
        def _():
            last_j = jnp.where(
                (n_mine - 1) % 2 == b, n_mine - 1, n_mine - 2
            )
            pltpu.make_async_copy(
                out_bufs[b],
                out_hbm.at[pl.ds(row0(last_j), _CHUNK), :],
                out_sems[b],
            ).wait()


@jax.jit
def kernel(atom):
    mesh = plsc.VectorSubcoreMesh(core_axis_name="c", subcore_axis_name="s")
    run = pl.kernel(
        _sc_body,
        out_type=jax.ShapeDtypeStruct((_N, _TOTAL), jnp.float32),
        mesh=mesh,
        scratch_types=[
            pltpu.VMEM((_CHUNK, _NFEAT), jnp.int32),
            pltpu.VMEM((_CHUNK, _NFEAT), jnp.int32),
            pltpu.VMEM((_CHUNK, _TOTAL), jnp.float32),
            pltpu.VMEM((_CHUNK, _TOTAL), jnp.float32),
            pltpu.SemaphoreType.DMA,
            pltpu.SemaphoreType.DMA,
            pltpu.SemaphoreType.DMA,
            pltpu.SemaphoreType.DMA,
        ],
        compiler_params=pltpu.CompilerParams(needs_layout_passes=False),
    )
    return run(atom.astype(jnp.int32))
